# baseline (device time: 16881 ns/iter reference)
import jax
import jax.numpy as jnp
from jax import lax
from jax.experimental import pallas as pl
from jax.experimental.pallas import tpu as pltpu

N_GLOBAL = 2048.0
EPS = 1e-5
C = 8
HALVES = 2


def kernel(x, gamma, beta):
    m, n_loc = x.shape
    R = m // C
    mh = m // HALVES
    cph = C // HALVES

    def body(x_hbm, g_hbm, b_hbm, out_hbm,
             x_vmem, out_vmem, g_vmem, b_vmem, stats_ref, peer_ref,
             in_sems, out_sems, gb_sems, send_sems, recv_sems):
        my_x = lax.axis_index("x")
        my_y = lax.axis_index("y")
        peer = (my_x, 1 - my_y)

        barrier_sem = pltpu.get_barrier_semaphore()
        pl.semaphore_signal(
            barrier_sem, inc=1, device_id=peer,
            device_id_type=pl.DeviceIdType.MESH,
        )

        g_dma = pltpu.make_async_copy(g_hbm, g_vmem, gb_sems.at[0])
        b_dma = pltpu.make_async_copy(b_hbm, b_vmem, gb_sems.at[1])
        g_dma.start()
        b_dma.start()
        in_dmas = []
        for c in range(C):
            dma = pltpu.make_async_copy(
                x_hbm.at[pl.ds(c * R, R), :],
                x_vmem.at[pl.ds(c * R, R), :],
                in_sems.at[c],
            )
            dma.start()
            in_dmas.append(dma)

        pl.semaphore_wait(barrier_sem, 1)

        rdmas = []
        for h in range(HALVES):
            for k in range(cph):
                c = h * cph + k
                in_dmas[c].wait()
                xv = x_vmem[pl.ds(c * R, R), :]
                stats_ref[h, 0, pl.ds(k * R, R)] = jnp.sum(xv, axis=1)
                stats_ref[h, 1, pl.ds(k * R, R)] = jnp.sum(xv * xv, axis=1)
            rdma = pltpu.make_async_remote_copy(
                src_ref=stats_ref.at[h],
                dst_ref=peer_ref.at[h],
                send_sem=send_sems.at[h],
                recv_sem=recv_sems.at[h],
                device_id=peer,
                device_id_type=pl.DeviceIdType.MESH,
            )
            rdma.start()
            rdmas.append(rdma)

        g_dma.wait()
        b_dma.wait()
        gv = g_vmem[:][None, :]
        bv = b_vmem[:][None, :]

        out_dmas = []
        for h in range(HALVES):
            rdmas[h].wait_recv()
            tot_s = stats_ref[h, 0, :] + peer_ref[h, 0, :]
            tot_sq = stats_ref[h, 1, :] + peer_ref[h, 1, :]
            mean = tot_s / N_GLOBAL
            var = tot_sq / N_GLOBAL - mean * mean
            inv = lax.rsqrt(var + EPS)
            for k in range(cph):
                c = h * cph + k
                xv = x_vmem[pl.ds(c * R, R), :]
                mu = mean[k * R:(k + 1) * R][:, None]
                iv = inv[k * R:(k + 1) * R][:, None]
                out_vmem[pl.ds(c * R, R), :] = gv * ((xv - mu) * iv) + bv
                dma = pltpu.make_async_copy(
                    out_vmem.at[pl.ds(c * R, R), :],
                    out_hbm.at[pl.ds(c * R, R), :],
                    out_sems.at[c],
                )
                dma.start()
                out_dmas.append(dma)

        for h in range(HALVES):
            rdmas[h].wait_send()
        for dma in out_dmas:
            dma.wait()

    return pl.pallas_call(
        body,
        out_shape=jax.ShapeDtypeStruct((m, n_loc), jnp.float32),
        in_specs=[
            pl.BlockSpec(memory_space=pl.ANY),
            pl.BlockSpec(memory_space=pl.ANY),
            pl.BlockSpec(memory_space=pl.ANY),
        ],
        out_specs=pl.BlockSpec(memory_space=pl.ANY),
        scratch_shapes=[
            pltpu.VMEM((m, n_loc), jnp.float32),
            pltpu.VMEM((m, n_loc), jnp.float32),
            pltpu.VMEM((n_loc,), jnp.float32),
            pltpu.VMEM((n_loc,), jnp.float32),
            pltpu.VMEM((HALVES, 2, mh), jnp.float32),
            pltpu.VMEM((HALVES, 2, mh), jnp.float32),
            pltpu.SemaphoreType.DMA((C,)),
            pltpu.SemaphoreType.DMA((C,)),
            pltpu.SemaphoreType.DMA((2,)),
            pltpu.SemaphoreType.DMA((HALVES,)),
            pltpu.SemaphoreType.DMA((HALVES,)),
        ],
        compiler_params=pltpu.CompilerParams(collective_id=0),
    )(x, gamma, beta)


# device time: 12347 ns/iter; 1.3672x vs baseline; 1.3672x over previous
import jax
import jax.numpy as jnp
from jax import lax
from jax.experimental import pallas as pl
from jax.experimental.pallas import tpu as pltpu

N_GLOBAL = 2048.0
EPS = 1e-5
C = 8
HALVES = 2


def kernel(x, gamma, beta):
    m, n_loc = x.shape
    R = m // C
    mh = m // HALVES
    cph = C // HALVES

    def body(x_hbm, g_hbm, b_hbm, out_ref,
             x_vmem, g_vmem, b_vmem, stats_ref, peer_ref,
             in_sems, gb_sems, send_sems, recv_sems):
        my_x = lax.axis_index("x")
        my_y = lax.axis_index("y")
        peer = (my_x, 1 - my_y)

        barrier_sem = pltpu.get_barrier_semaphore()
        pl.semaphore_signal(
            barrier_sem, inc=1, device_id=peer,
            device_id_type=pl.DeviceIdType.MESH,
        )

        g_dma = pltpu.make_async_copy(g_hbm, g_vmem, gb_sems.at[0])
        b_dma = pltpu.make_async_copy(b_hbm, b_vmem, gb_sems.at[1])
        g_dma.start()
        b_dma.start()
        in_dmas = []
        for c in range(C):
            dma = pltpu.make_async_copy(
                x_hbm.at[pl.ds(c * R, R), :],
                x_vmem.at[pl.ds(c * R, R), :],
                in_sems.at[c],
            )
            dma.start()
            in_dmas.append(dma)

        pl.semaphore_wait(barrier_sem, 1)

        rdmas = []
        for h in range(HALVES):
            for k in range(cph):
                c = h * cph + k
                in_dmas[c].wait()
                xv = x_vmem[pl.ds(c * R, R), :]
                stats_ref[h, 0, pl.ds(k * R, R)] = jnp.sum(xv, axis=1)
                stats_ref[h, 1, pl.ds(k * R, R)] = jnp.sum(xv * xv, axis=1)
            rdma = pltpu.make_async_remote_copy(
                src_ref=stats_ref.at[h],
                dst_ref=peer_ref.at[h],
                send_sem=send_sems.at[h],
                recv_sem=recv_sems.at[h],
                device_id=peer,
                device_id_type=pl.DeviceIdType.MESH,
            )
            rdma.start()
            rdmas.append(rdma)

        g_dma.wait()
        b_dma.wait()
        gv = g_vmem[:][None, :]
        bv = b_vmem[:][None, :]

        for h in range(HALVES):
            rdmas[h].wait_recv()
            tot_s = stats_ref[h, 0, :] + peer_ref[h, 0, :]
            tot_sq = stats_ref[h, 1, :] + peer_ref[h, 1, :]
            mean = tot_s / N_GLOBAL
            var = tot_sq / N_GLOBAL - mean * mean
            inv = lax.rsqrt(var + EPS)
            for k in range(cph):
                c = h * cph + k
                xv = x_vmem[pl.ds(c * R, R), :]
                mu = mean[k * R:(k + 1) * R][:, None]
                iv = inv[k * R:(k + 1) * R][:, None]
                out_ref[pl.ds(c * R, R), :] = gv * ((xv - mu) * iv) + bv

        for h in range(HALVES):
            rdmas[h].wait_send()

    return pl.pallas_call(
        body,
        out_shape=jax.ShapeDtypeStruct((m, n_loc), jnp.float32),
        in_specs=[
            pl.BlockSpec(memory_space=pltpu.MemorySpace.HBM),
            pl.BlockSpec(memory_space=pltpu.MemorySpace.HBM),
            pl.BlockSpec(memory_space=pltpu.MemorySpace.HBM),
        ],
        out_specs=pl.BlockSpec(memory_space=pltpu.MemorySpace.VMEM),
        scratch_shapes=[
            pltpu.VMEM((m, n_loc), jnp.float32),
            pltpu.VMEM((n_loc,), jnp.float32),
            pltpu.VMEM((n_loc,), jnp.float32),
            pltpu.VMEM((HALVES, 2, mh), jnp.float32),
            pltpu.VMEM((HALVES, 2, mh), jnp.float32),
            pltpu.SemaphoreType.DMA((C,)),
            pltpu.SemaphoreType.DMA((2,)),
            pltpu.SemaphoreType.DMA((HALVES,)),
            pltpu.SemaphoreType.DMA((HALVES,)),
        ],
        compiler_params=pltpu.CompilerParams(collective_id=0),
    )(
        pltpu.with_memory_space_constraint(x, pltpu.MemorySpace.HBM),
        pltpu.with_memory_space_constraint(gamma, pltpu.MemorySpace.HBM),
        pltpu.with_memory_space_constraint(beta, pltpu.MemorySpace.HBM),
    )
